# baseline (speedup 1.0000x reference)
import jax
import jax.numpy as jnp
from jax.experimental import pallas as pl
from jax.experimental.pallas import tpu as pltpu

_K = 32
_S = 250
_B = 2
_NDROP = 4


def _body(drop_ref, x_ref, o_ref, zblk_ref, csem, zsem):
    copies = []
    for b in range(_B):
        for i in range(8):
            d0 = i * _K
            dn = min(_K, _S - d0)
            c = pltpu.make_async_copy(
                x_ref.at[b, pl.ds(d0, dn)], o_ref.at[b, pl.ds(d0, dn)], csem
            )
            c.start()
            copies.append(c)
    for c in copies:
        c.wait()

    for n in range(_NDROP):
        p = drop_ref[n]
        pd, ph, pw = p // 64, (p // 8) % 8, p % 8
        d0 = jnp.minimum(pd * _K, _S - _K)
        wmask = None
        for b in range(_B):

            @pl.when(ph != 7)
            def _(pd=pd, ph=ph, pw=pw, d0=d0, b=b):
                h0 = pl.multiple_of(ph * _K, 8)
                sl = (b, pl.ds(d0, _K), pl.ds(h0, _K), slice(None))
                small = zblk_ref.at[:, pl.ds(0, _K), :]
                cin = pltpu.make_async_copy(o_ref.at[sl], small, zsem)
                cin.start()
                cin.wait()
                di = d0 + jax.lax.broadcasted_iota(jnp.int32, (_K, 1, 1), 0)
                wi = jax.lax.broadcasted_iota(jnp.int32, (1, 1, _S), 2)
                m = (
                    (di >= pd * _K)
                    & (wi >= pw * _K)
                    & (wi < pw * _K + _K)
                )
                small[...] = jnp.where(m, 0.0, small[...])
                cout = pltpu.make_async_copy(small, o_ref.at[sl], zsem)
                cout.start()
                cout.wait()

            @pl.when(ph == 7)
            def _(pd=pd, ph=ph, pw=pw, d0=d0, b=b):
                sl = (b, pl.ds(d0, _K), slice(None), slice(None))
                cin = pltpu.make_async_copy(o_ref.at[sl], zblk_ref, zsem)
                cin.start()
                cin.wait()
                di = d0 + jax.lax.broadcasted_iota(jnp.int32, (_K, 1, 1), 0)
                hi = jax.lax.broadcasted_iota(jnp.int32, (1, _S, 1), 1)
                wi = jax.lax.broadcasted_iota(jnp.int32, (1, 1, _S), 2)
                m = (
                    (di >= pd * _K)
                    & (hi >= ph * _K)
                    & (wi >= pw * _K)
                    & (wi < pw * _K + _K)
                )
                zblk_ref[...] = jnp.where(m, 0.0, zblk_ref[...])
                cout = pltpu.make_async_copy(zblk_ref, o_ref.at[sl], zsem)
                cout.start()
                cout.wait()

        del wmask


def kernel(x, drop_idx):
    xs = x.reshape(_B, _S, _S, _S)
    out = pl.pallas_call(
        _body,
        in_specs=[
            pl.BlockSpec(memory_space=pltpu.SMEM),
            pl.BlockSpec(memory_space=pl.ANY),
        ],
        out_specs=pl.BlockSpec(memory_space=pl.ANY),
        out_shape=jax.ShapeDtypeStruct((_B, _S, _S, _S), jnp.float32),
        scratch_shapes=[
            pltpu.VMEM((_K, _S, _S), jnp.float32),
            pltpu.SemaphoreType.DMA,
            pltpu.SemaphoreType.DMA,
        ],
    )(drop_idx.astype(jnp.int32), xs)
    return out.reshape(x.shape)


# speedup vs baseline: 12.6772x; 12.6772x over previous
import jax
import jax.numpy as jnp
from jax.experimental import pallas as pl
from jax.experimental.pallas import tpu as pltpu

_K = 32
_S = 250
_NDROP = 4


def _body(drop_ref, x_ref, o_ref):
    bd = pl.program_id(1)
    bh = pl.program_id(2)
    hits = []
    for n in range(_NDROP):
        p = drop_ref[n]
        hits.append((p // 64 == bd) & ((p // 8) % 8 == bh))
    any_hit = hits[0] | hits[1] | hits[2] | hits[3]

    @pl.when(jnp.logical_not(any_hit))
    def _():
        o_ref[...] = x_ref[...]

    @pl.when(any_hit)
    def _():
        wp = jax.lax.broadcasted_iota(jnp.int32, (1, 1, 1, _S), 3) // _K
        mask = None
        for n in range(_NDROP):
            m = hits[n] & (drop_ref[n] % 8 == wp)
            mask = m if mask is None else mask | m
        o_ref[...] = jnp.where(mask, 0.0, x_ref[...])


def kernel(x, drop_idx):
    B = x.shape[0]
    xs = x.reshape(B, _S, _S, _S)
    nblk = (_S + _K - 1) // _K
    out = pl.pallas_call(
        _body,
        grid=(B, nblk, nblk),
        in_specs=[
            pl.BlockSpec(memory_space=pltpu.SMEM),
            pl.BlockSpec((1, _K, _K, _S), lambda b, i, j: (b, i, j, 0)),
        ],
        out_specs=pl.BlockSpec((1, _K, _K, _S), lambda b, i, j: (b, i, j, 0)),
        out_shape=jax.ShapeDtypeStruct((B, _S, _S, _S), jnp.float32),
        compiler_params=pltpu.CompilerParams(
            dimension_semantics=("parallel", "parallel", "parallel"),
        ),
    )(drop_idx.astype(jnp.int32), xs)
    return out.reshape(x.shape)
